# baseline (speedup 1.0000x reference)
import functools

import jax
import jax.numpy as jnp
import numpy as np
from jax import lax
from jax.experimental import pallas as pl
from jax.experimental.pallas import tpu as pltpu
from jax.experimental.pallas import tpu_sc as plsc

N = 10000
E = 160000
T = 3
R = 5
L = 2
DM = 128
H = 8
DH = 16

NROW = 10240
TRASH = NROW - 1
NW = 32
EPAD = 163840
EPT = EPAD // NW
C = 128
G = EPT // C
NB = 256
NBLK = NROW // NB
EB = 1024
EBLK = EPAD // EB

_mesh = plsc.VectorSubcoreMesh(core_axis_name="c", subcore_axis_name="s")

_repc = np.zeros((DM, DM), np.float32)
for _j in range(H):
    _repc[_j * DH:(_j + 1) * DH, _j] = 1.0
_REPC = _repc
_REPR = _repc[:, :16].T.copy()


def _dense1_body(h_ref, oh_ref, wq_ref, wkr_ref, wvr_ref, q_ref, kr_ref, vr_ref):
    x = h_ref[...]
    q = jnp.zeros((NB, DM), jnp.float32)
    kr = [jnp.zeros((NB, DM), jnp.float32) for _ in range(R)]
    vr = [jnp.zeros((NB, DM), jnp.float32) for _ in range(R)]
    for t in range(T):
        m = oh_ref[:, t:t + 1]
        q = q + m * jnp.dot(x, wq_ref[t], preferred_element_type=jnp.float32)
        for r in range(R):
            kr[r] = kr[r] + m * jnp.dot(x, wkr_ref[t, r], preferred_element_type=jnp.float32)
            vr[r] = vr[r] + m * jnp.dot(x, wvr_ref[t, r], preferred_element_type=jnp.float32)
    q_ref[...] = q
    kr_ref[...] = jnp.concatenate(kr, axis=1)
    vr_ref[...] = jnp.concatenate(vr, axis=1)


def _dense1(h, ohf, wq, wkr, wvr):
    return pl.pallas_call(
        _dense1_body,
        grid=(NBLK,),
        in_specs=[
            pl.BlockSpec((NB, DM), lambda i: (i, 0)),
            pl.BlockSpec((NB, 8), lambda i: (i, 0)),
            pl.BlockSpec((T, DM, DM), lambda i: (0, 0, 0)),
            pl.BlockSpec((T, R, DM, DM), lambda i: (0, 0, 0, 0)),
            pl.BlockSpec((T, R, DM, DM), lambda i: (0, 0, 0, 0)),
        ],
        out_specs=[
            pl.BlockSpec((NB, DM), lambda i: (i, 0)),
            pl.BlockSpec((NB, R * DM), lambda i: (i, 0)),
            pl.BlockSpec((NB, R * DM), lambda i: (i, 0)),
        ],
        out_shape=[
            jax.ShapeDtypeStruct((NROW, DM), jnp.float32),
            jax.ShapeDtypeStruct((NROW, R * DM), jnp.float32),
            jax.ShapeDtypeStruct((NROW, R * DM), jnp.float32),
        ],
    )(h, ohf, wq, wkr, wvr)


def _edgef_body(kre_ref, qe_ref, vre_ref, repc_ref, mask_ref, repr_ref,
                aexp_ref, msg_ref):
    prod = kre_ref[...] * qe_ref[...]
    s = jnp.dot(prod, repc_ref[...], preferred_element_type=jnp.float32)
    ae = jnp.exp(s) * mask_ref[...]
    aexp_ref[...] = ae
    sa128 = jnp.dot(ae[:, :16], repr_ref[...],
                    preferred_element_type=jnp.float32)
    msg_ref[...] = vre_ref[...] * sa128


def _edgef(kre, qe, vre, repc, mask8, repr_):
    return pl.pallas_call(
        _edgef_body,
        grid=(EBLK,),
        in_specs=[
            pl.BlockSpec((EB, DM), lambda i: (i, 0)),
            pl.BlockSpec((EB, DM), lambda i: (i, 0)),
            pl.BlockSpec((EB, DM), lambda i: (i, 0)),
            pl.BlockSpec((DM, DM), lambda i: (0, 0)),
            pl.BlockSpec((1, DM), lambda i: (0, 0)),
            pl.BlockSpec((16, DM), lambda i: (0, 0)),
        ],
        out_specs=[
            pl.BlockSpec((EB, DM), lambda i: (i, 0)),
            pl.BlockSpec((EB, DM), lambda i: (i, 0)),
        ],
        out_shape=[
            jax.ShapeDtypeStruct((EPAD, DM), jnp.float32),
            jax.ShapeDtypeStruct((EPAD, DM), jnp.float32),
        ],
    )(kre, qe, vre, repc, mask8, repr_)


def _dense2_body(hg0_ref, hg1_ref, dn0_ref, dn1_ref, repr_ref, h_ref, oh_ref,
                 al_ref, wa_ref, lng_ref, lnb_ref, out_ref):
    den = dn0_ref[:, :16] + dn1_ref[:, :16]
    recip = 1.0 / jnp.maximum(den, 1e-9)
    r128 = jnp.dot(recip, repr_ref[...], preferred_element_type=jnp.float32)
    hagg = (hg0_ref[...] + hg1_ref[...]) * r128
    x = h_ref[...]
    hc = jnp.zeros((NB, DM), jnp.float32)
    for t in range(T):
        m = oh_ref[:, t:t + 1]
        hc = hc + m * jnp.dot(hagg, wa_ref[t], preferred_element_type=jnp.float32)
    alpha = al_ref[:, 0:1]
    hc = hc * alpha + x * (1.0 - alpha)
    mu = jnp.mean(hc, axis=-1, keepdims=True)
    var = jnp.mean((hc - mu) ** 2, axis=-1, keepdims=True)
    hn = (hc - mu) * lax.rsqrt(var + 1e-5) * lng_ref[...] + lnb_ref[...]
    out_ref[...] = hn + x


def _dense2(hg0, hg1, dn0, dn1, repr_, h, ohf, alcol, wa, lng, lnb):
    return pl.pallas_call(
        _dense2_body,
        grid=(NBLK,),
        in_specs=[
            pl.BlockSpec((NB, DM), lambda i: (i, 0)),
            pl.BlockSpec((NB, DM), lambda i: (i, 0)),
            pl.BlockSpec((NB, DM), lambda i: (i, 0)),
            pl.BlockSpec((NB, DM), lambda i: (i, 0)),
            pl.BlockSpec((16, DM), lambda i: (0, 0)),
            pl.BlockSpec((NB, DM), lambda i: (i, 0)),
            pl.BlockSpec((NB, 8), lambda i: (i, 0)),
            pl.BlockSpec((NB, 8), lambda i: (i, 0)),
            pl.BlockSpec((T, DM, DM), lambda i: (0, 0, 0)),
            pl.BlockSpec((1, DM), lambda i: (0, 0)),
            pl.BlockSpec((1, DM), lambda i: (0, 0)),
        ],
        out_specs=pl.BlockSpec((NB, DM), lambda i: (i, 0)),
        out_shape=jax.ShapeDtypeStruct((NROW, DM), jnp.float32),
    )(hg0, hg1, dn0, dn1, repr_, h, ohf, alcol, wa, lng, lnb)


def _wid():
    return lax.axis_index("s") * 2 + lax.axis_index("c")


@functools.partial(
    pl.kernel, mesh=_mesh,
    out_type=jax.ShapeDtypeStruct((NROW, DM), jnp.float32),
    scratch_types=[
        pltpu.VMEM((80,), jnp.int32),
        pltpu.VMEM((80, DM), jnp.float32),
        pltpu.SemaphoreType.DMA,
    ],
)
def _sc_gather_rows(table_hbm, idx_hbm, out_hbm, idxv, rowsv, sem):
    w = _wid()

    def chunk(g, _):
        base = w * 320 + g * 80
        pltpu.sync_copy(idx_hbm.at[pl.ds(base, 80)], idxv)
        pltpu.async_copy(table_hbm.at[idxv], rowsv, sem).wait()
        pltpu.sync_copy(rowsv, out_hbm.at[pl.ds(base, 80)])
        return 0
    lax.fori_loop(0, 4, chunk, 0, unroll=False)


@functools.partial(
    pl.kernel, mesh=_mesh,
    out_type=[
        jax.ShapeDtypeStruct((EPAD, DM), jnp.float32),
        jax.ShapeDtypeStruct((EPAD, DM), jnp.float32),
        jax.ShapeDtypeStruct((EPAD, DM), jnp.float32),
    ],
    scratch_types=[
        pltpu.VMEM((C,), jnp.int32),
        pltpu.VMEM((C,), jnp.int32),
        pltpu.VMEM((C, DM), jnp.float32),
        pltpu.VMEM((C, DM), jnp.float32),
        pltpu.VMEM((C, DM), jnp.float32),
        pltpu.SemaphoreType.DMA,
        pltpu.SemaphoreType.DMA,
        pltpu.SemaphoreType.DMA,
    ],
)
def _sc_gather3(srcet_hbm, dst_hbm, kr_hbm, q_hbm, vr_hbm,
                kre_out, qe_out, vre_out, idxa, idxd, krv, qv, vrv,
                sem1, sem2, sem3):
    w = _wid()

    def chunk(g, _):
        base = (w * G + g) * C
        pltpu.sync_copy(srcet_hbm.at[pl.ds(base, C)], idxa)
        pltpu.sync_copy(dst_hbm.at[pl.ds(base, C)], idxd)
        cp1 = pltpu.async_copy(kr_hbm.at[idxa], krv, sem1)
        cp2 = pltpu.async_copy(q_hbm.at[idxd], qv, sem2)
        cp3 = pltpu.async_copy(vr_hbm.at[idxa], vrv, sem3)
        cp1.wait()
        pltpu.sync_copy(krv, kre_out.at[pl.ds(base, C)])
        cp2.wait()
        pltpu.sync_copy(qv, qe_out.at[pl.ds(base, C)])
        cp3.wait()
        pltpu.sync_copy(vrv, vre_out.at[pl.ds(base, C)])
        return 0
    lax.fori_loop(0, G, chunk, 0, unroll=False)


@functools.partial(
    pl.kernel, mesh=_mesh,
    out_type=jax.ShapeDtypeStruct((2, NROW, DM), jnp.float32),
    scratch_types=[
        pltpu.VMEM((C,), jnp.int32),
        pltpu.VMEM((C, DM), jnp.float32),
        pltpu.VMEM_SHARED((NROW, DM), jnp.float32),
    ],
)
def _sc_scatter128(dst_hbm, rows_hbm, zeros_hbm, out_hbm, idxd, rowv, accsh):
    c = lax.axis_index("c")
    s = lax.axis_index("s")
    w = _wid()

    @pl.when(s == 0)
    def _init():
        pltpu.sync_copy(zeros_hbm, accsh)
    plsc.subcore_barrier()

    def chunk(g, _):
        base = (w * G + g) * C
        pltpu.sync_copy(dst_hbm.at[pl.ds(base, C)], idxd)
        pltpu.sync_copy(rows_hbm.at[pl.ds(base, C)], rowv)
        pltpu.sync_copy(rowv, accsh.at[idxd], add=True)
        return 0
    lax.fori_loop(0, G, chunk, 0, unroll=False)

    plsc.subcore_barrier()

    @pl.when(s == 0)
    def _dump():
        pltpu.sync_copy(accsh, out_hbm.at[c])


def kernel(node_type, edge_index, edge_type, embed, Wk, Wq, Wv, Wa, Watt,
           Wmsg, pri, skip, ln_g, ln_b):
    node_type = node_type.astype(jnp.int32)
    edge_type = edge_type.astype(jnp.int32)
    src = edge_index[0].astype(jnp.int32)
    dst = edge_index[1].astype(jnp.int32)

    oh = jax.nn.one_hot(node_type, T, dtype=jnp.int32)
    local = jnp.cumsum(oh, axis=0)[jnp.arange(N), node_type] - 1
    flatidx = jnp.zeros((NROW,), jnp.int32).at[:N].set(node_type * N + local)
    ohf = jnp.zeros((NROW, 8), jnp.float32).at[:N, :T].set(
        oh.astype(jnp.float32))
    alpha = jax.nn.sigmoid(skip)
    alcol = jnp.zeros((L, NROW, 8), jnp.float32).at[:, :N, 0].set(
        alpha[:, node_type])
    srcet = jnp.full((EPAD,), TRASH * R, jnp.int32).at[:E].set(
        src * R + edge_type)
    dste = jnp.full((EPAD,), TRASH, jnp.int32).at[:E].set(dst)
    zeros128 = jnp.zeros((NROW, DM), jnp.float32)
    zeros16 = jnp.zeros((NROW, 16), jnp.float32)
    repc = jnp.asarray(_REPC)
    repr_ = jnp.asarray(_REPR)
    mask8 = jnp.zeros((1, DM), jnp.float32).at[0, :H].set(1.0)

    watt_s = Watt * (pri / jnp.sqrt(jnp.float32(DH)))[:, :, :, None, None]
    wk4 = Wk.reshape(L, T, DM, H, DH)
    wv4 = Wv.reshape(L, T, DM, H, DH)
    wkr = jnp.einsum('ltdhe,lhref->ltrdhf', wk4, watt_s).reshape(L, T, R, DM, DM)
    wvr = jnp.einsum('ltdhe,lhref->ltrdhf', wv4, Wmsg).reshape(L, T, R, DM, DM)

    h = _sc_gather_rows(embed.reshape(T * N, DM), flatidx)

    for l in range(L):
        q, krcat, vrcat = _dense1(h, ohf, Wq[l], wkr[l], wvr[l])
        kr = krcat.reshape(NROW * R, DM)
        vr = vrcat.reshape(NROW * R, DM)
        kre, qe, vre = _sc_gather3(srcet, dste, kr, q, vr)
        aexp, msg = _edgef(kre, qe, vre, repc, mask8, repr_)
        denp = _sc_scatter128(dste, aexp, zeros128)
        haggp = _sc_scatter128(dste, msg, zeros128)
        h = _dense2(haggp[0], haggp[1], denp[0], denp[1], repr_, h, ohf,
                    alcol[l], Wa[l],
                    ln_g[l].reshape(1, DM), ln_b[l].reshape(1, DM))
    return h[:N]


# speedup vs baseline: 1.0417x; 1.0417x over previous
import functools

import jax
import jax.numpy as jnp
import numpy as np
from jax import lax
from jax.experimental import pallas as pl
from jax.experimental.pallas import tpu as pltpu
from jax.experimental.pallas import tpu_sc as plsc

N = 10000
E = 160000
T = 3
R = 5
L = 2
DM = 128
H = 8
DH = 16

NROW = 10240
TRASH = NROW - 1
NW = 32
EPAD = 163840
EPT = EPAD // NW
C = 128
G = EPT // C
NB = 256
NBLK = NROW // NB
EB = 1024
EBLK = EPAD // EB

_mesh = plsc.VectorSubcoreMesh(core_axis_name="c", subcore_axis_name="s")

_repc = np.zeros((DM, DM), np.float32)
for _j in range(H):
    _repc[_j * DH:(_j + 1) * DH, _j] = 1.0
_REPC = _repc
_REPR = _repc[:, :16].T.copy()


def _dense1_body(h_ref, oh_ref, wq_ref, wkr_ref, wvr_ref, q_ref, kr_ref, vr_ref):
    x = h_ref[...]
    q = jnp.zeros((NB, DM), jnp.float32)
    kr = [jnp.zeros((NB, DM), jnp.float32) for _ in range(R)]
    vr = [jnp.zeros((NB, DM), jnp.float32) for _ in range(R)]
    for t in range(T):
        m = oh_ref[:, t:t + 1]
        q = q + m * jnp.dot(x, wq_ref[t], preferred_element_type=jnp.float32)
        for r in range(R):
            kr[r] = kr[r] + m * jnp.dot(x, wkr_ref[t, r], preferred_element_type=jnp.float32)
            vr[r] = vr[r] + m * jnp.dot(x, wvr_ref[t, r], preferred_element_type=jnp.float32)
    q_ref[...] = q
    kr_ref[...] = jnp.concatenate(kr, axis=1)
    vr_ref[...] = jnp.concatenate(vr, axis=1)


def _dense1(h, ohf, wq, wkr, wvr):
    return pl.pallas_call(
        _dense1_body,
        grid=(NBLK,),
        in_specs=[
            pl.BlockSpec((NB, DM), lambda i: (i, 0)),
            pl.BlockSpec((NB, 8), lambda i: (i, 0)),
            pl.BlockSpec((T, DM, DM), lambda i: (0, 0, 0)),
            pl.BlockSpec((T, R, DM, DM), lambda i: (0, 0, 0, 0)),
            pl.BlockSpec((T, R, DM, DM), lambda i: (0, 0, 0, 0)),
        ],
        out_specs=[
            pl.BlockSpec((NB, DM), lambda i: (i, 0)),
            pl.BlockSpec((NB, R * DM), lambda i: (i, 0)),
            pl.BlockSpec((NB, R * DM), lambda i: (i, 0)),
        ],
        out_shape=[
            jax.ShapeDtypeStruct((NROW, DM), jnp.float32),
            jax.ShapeDtypeStruct((NROW, R * DM), jnp.float32),
            jax.ShapeDtypeStruct((NROW, R * DM), jnp.float32),
        ],
    )(h, ohf, wq, wkr, wvr)


def _edgef_body(kre_ref, qe_ref, vre_ref, repc_ref, mask_ref, repr_ref,
                aexp_ref, msg_ref):
    prod = kre_ref[...] * qe_ref[...]
    s = jnp.dot(prod, repc_ref[...], preferred_element_type=jnp.float32)
    ae = jnp.exp(s) * mask_ref[...]
    aexp_ref[...] = ae
    sa128 = jnp.dot(ae[:, :16], repr_ref[...],
                    preferred_element_type=jnp.float32)
    msg_ref[...] = vre_ref[...] * sa128


def _edgef(kre, qe, vre, repc, mask8, repr_):
    return pl.pallas_call(
        _edgef_body,
        grid=(EBLK,),
        in_specs=[
            pl.BlockSpec((EB, DM), lambda i: (i, 0)),
            pl.BlockSpec((EB, DM), lambda i: (i, 0)),
            pl.BlockSpec((EB, DM), lambda i: (i, 0)),
            pl.BlockSpec((DM, DM), lambda i: (0, 0)),
            pl.BlockSpec((1, DM), lambda i: (0, 0)),
            pl.BlockSpec((16, DM), lambda i: (0, 0)),
        ],
        out_specs=[
            pl.BlockSpec((EB, DM), lambda i: (i, 0)),
            pl.BlockSpec((EB, DM), lambda i: (i, 0)),
        ],
        out_shape=[
            jax.ShapeDtypeStruct((EPAD, DM), jnp.float32),
            jax.ShapeDtypeStruct((EPAD, DM), jnp.float32),
        ],
    )(kre, qe, vre, repc, mask8, repr_)


def _dense2_body(hg0_ref, hg1_ref, dn0_ref, dn1_ref, repr_ref, h_ref, oh_ref,
                 al_ref, wa_ref, lng_ref, lnb_ref, out_ref):
    den = dn0_ref[:, :16] + dn1_ref[:, :16]
    recip = 1.0 / jnp.maximum(den, 1e-9)
    r128 = jnp.dot(recip, repr_ref[...], preferred_element_type=jnp.float32)
    hagg = (hg0_ref[...] + hg1_ref[...]) * r128
    x = h_ref[...]
    hc = jnp.zeros((NB, DM), jnp.float32)
    for t in range(T):
        m = oh_ref[:, t:t + 1]
        hc = hc + m * jnp.dot(hagg, wa_ref[t], preferred_element_type=jnp.float32)
    alpha = al_ref[:, 0:1]
    hc = hc * alpha + x * (1.0 - alpha)
    mu = jnp.mean(hc, axis=-1, keepdims=True)
    var = jnp.mean((hc - mu) ** 2, axis=-1, keepdims=True)
    hn = (hc - mu) * lax.rsqrt(var + 1e-5) * lng_ref[...] + lnb_ref[...]
    out_ref[...] = hn + x


def _dense2(hg0, hg1, dn0, dn1, repr_, h, ohf, alcol, wa, lng, lnb):
    return pl.pallas_call(
        _dense2_body,
        grid=(NBLK,),
        in_specs=[
            pl.BlockSpec((NB, DM), lambda i: (i, 0)),
            pl.BlockSpec((NB, DM), lambda i: (i, 0)),
            pl.BlockSpec((NB, DM), lambda i: (i, 0)),
            pl.BlockSpec((NB, DM), lambda i: (i, 0)),
            pl.BlockSpec((16, DM), lambda i: (0, 0)),
            pl.BlockSpec((NB, DM), lambda i: (i, 0)),
            pl.BlockSpec((NB, 8), lambda i: (i, 0)),
            pl.BlockSpec((NB, 8), lambda i: (i, 0)),
            pl.BlockSpec((T, DM, DM), lambda i: (0, 0, 0)),
            pl.BlockSpec((1, DM), lambda i: (0, 0)),
            pl.BlockSpec((1, DM), lambda i: (0, 0)),
        ],
        out_specs=pl.BlockSpec((NB, DM), lambda i: (i, 0)),
        out_shape=jax.ShapeDtypeStruct((NROW, DM), jnp.float32),
    )(hg0, hg1, dn0, dn1, repr_, h, ohf, alcol, wa, lng, lnb)


def _wid():
    return lax.axis_index("s") * 2 + lax.axis_index("c")


@functools.partial(
    pl.kernel, mesh=_mesh,
    out_type=jax.ShapeDtypeStruct((NROW, DM), jnp.float32),
    scratch_types=[
        pltpu.VMEM((80,), jnp.int32),
        pltpu.VMEM((80, DM), jnp.float32),
        pltpu.SemaphoreType.DMA,
    ],
)
def _sc_gather_rows(table_hbm, idx_hbm, out_hbm, idxv, rowsv, sem):
    w = _wid()

    def chunk(g, _):
        base = w * 320 + g * 80
        pltpu.sync_copy(idx_hbm.at[pl.ds(base, 80)], idxv)
        pltpu.async_copy(table_hbm.at[idxv], rowsv, sem).wait()
        pltpu.sync_copy(rowsv, out_hbm.at[pl.ds(base, 80)])
        return 0
    lax.fori_loop(0, 4, chunk, 0, unroll=False)


@functools.partial(
    pl.kernel, mesh=_mesh,
    out_type=[
        jax.ShapeDtypeStruct((EPAD, DM), jnp.float32),
        jax.ShapeDtypeStruct((EPAD, DM), jnp.float32),
        jax.ShapeDtypeStruct((EPAD, DM), jnp.float32),
    ],
    scratch_types=[
        pltpu.VMEM((2, C), jnp.int32),
        pltpu.VMEM((2, C), jnp.int32),
        pltpu.VMEM((2, C, DM), jnp.float32),
        pltpu.VMEM((2, C, DM), jnp.float32),
        pltpu.VMEM((2, C, DM), jnp.float32),
        pltpu.SemaphoreType.DMA,
        pltpu.SemaphoreType.DMA,
    ],
)
def _sc_gather3(srcet_hbm, dst_hbm, kr_hbm, q_hbm, vr_hbm,
                kre_out, qe_out, vre_out, idxa, idxd, krv, qv, vrv,
                semg, semw):
    w = _wid()

    def pair(g2, _):
        base0 = (w * G + 2 * g2) * C
        base1 = base0 + C
        pltpu.sync_copy(srcet_hbm.at[pl.ds(base0, C)], idxa.at[0])
        pltpu.sync_copy(dst_hbm.at[pl.ds(base0, C)], idxd.at[0])
        pltpu.sync_copy(srcet_hbm.at[pl.ds(base1, C)], idxa.at[1])
        pltpu.sync_copy(dst_hbm.at[pl.ds(base1, C)], idxd.at[1])
        cps = [
            pltpu.async_copy(kr_hbm.at[idxa.at[0]], krv.at[0], semg),
            pltpu.async_copy(q_hbm.at[idxd.at[0]], qv.at[0], semg),
            pltpu.async_copy(vr_hbm.at[idxa.at[0]], vrv.at[0], semg),
            pltpu.async_copy(kr_hbm.at[idxa.at[1]], krv.at[1], semg),
            pltpu.async_copy(q_hbm.at[idxd.at[1]], qv.at[1], semg),
            pltpu.async_copy(vr_hbm.at[idxa.at[1]], vrv.at[1], semg),
        ]
        for cp in cps:
            cp.wait()
        wps = [
            pltpu.async_copy(krv.at[0], kre_out.at[pl.ds(base0, C)], semw),
            pltpu.async_copy(qv.at[0], qe_out.at[pl.ds(base0, C)], semw),
            pltpu.async_copy(vrv.at[0], vre_out.at[pl.ds(base0, C)], semw),
            pltpu.async_copy(krv.at[1], kre_out.at[pl.ds(base1, C)], semw),
            pltpu.async_copy(qv.at[1], qe_out.at[pl.ds(base1, C)], semw),
            pltpu.async_copy(vrv.at[1], vre_out.at[pl.ds(base1, C)], semw),
        ]
        for wp in wps:
            wp.wait()
        return 0
    lax.fori_loop(0, G // 2, pair, 0, unroll=False)


@functools.partial(
    pl.kernel, mesh=_mesh,
    out_type=jax.ShapeDtypeStruct((2, NROW, DM), jnp.float32),
    scratch_types=[
        pltpu.VMEM((2, C), jnp.int32),
        pltpu.VMEM((2, C, DM), jnp.float32),
        pltpu.VMEM_SHARED((NROW, DM), jnp.float32),
        pltpu.SemaphoreType.DMA,
        pltpu.SemaphoreType.DMA,
    ],
)
def _sc_scatter128(dst_hbm, rows_hbm, zeros_hbm, out_hbm, idxd, rowv, accsh,
                   semr, sems):
    c = lax.axis_index("c")
    s = lax.axis_index("s")
    w = _wid()

    @pl.when(s == 0)
    def _init():
        pltpu.sync_copy(zeros_hbm, accsh)
    plsc.subcore_barrier()

    def pair(g2, _):
        base0 = (w * G + 2 * g2) * C
        base1 = base0 + C
        pltpu.sync_copy(dst_hbm.at[pl.ds(base0, C)], idxd.at[0])
        pltpu.sync_copy(dst_hbm.at[pl.ds(base1, C)], idxd.at[1])
        cp0 = pltpu.async_copy(rows_hbm.at[pl.ds(base0, C)], rowv.at[0], semr)
        cp1 = pltpu.async_copy(rows_hbm.at[pl.ds(base1, C)], rowv.at[1], semr)
        cp0.wait()
        sc0 = pltpu.async_copy(rowv.at[0], accsh.at[idxd.at[0]], sems,
                               add=True)
        cp1.wait()
        sc1 = pltpu.async_copy(rowv.at[1], accsh.at[idxd.at[1]], sems,
                               add=True)
        sc0.wait()
        sc1.wait()
        return 0
    lax.fori_loop(0, G // 2, pair, 0, unroll=False)

    plsc.subcore_barrier()

    @pl.when(s == 0)
    def _dump():
        pltpu.sync_copy(accsh, out_hbm.at[c])


def kernel(node_type, edge_index, edge_type, embed, Wk, Wq, Wv, Wa, Watt,
           Wmsg, pri, skip, ln_g, ln_b):
    node_type = node_type.astype(jnp.int32)
    edge_type = edge_type.astype(jnp.int32)
    src = edge_index[0].astype(jnp.int32)
    dst = edge_index[1].astype(jnp.int32)

    oh = jax.nn.one_hot(node_type, T, dtype=jnp.int32)
    local = jnp.cumsum(oh, axis=0)[jnp.arange(N), node_type] - 1
    flatidx = jnp.zeros((NROW,), jnp.int32).at[:N].set(node_type * N + local)
    ohf = jnp.zeros((NROW, 8), jnp.float32).at[:N, :T].set(
        oh.astype(jnp.float32))
    alpha = jax.nn.sigmoid(skip)
    alcol = jnp.zeros((L, NROW, 8), jnp.float32).at[:, :N, 0].set(
        alpha[:, node_type])
    srcet = jnp.full((EPAD,), TRASH * R, jnp.int32).at[:E].set(
        src * R + edge_type)
    dste = jnp.full((EPAD,), TRASH, jnp.int32).at[:E].set(dst)
    zeros128 = jnp.zeros((NROW, DM), jnp.float32)
    zeros16 = jnp.zeros((NROW, 16), jnp.float32)
    repc = jnp.asarray(_REPC)
    repr_ = jnp.asarray(_REPR)
    mask8 = jnp.zeros((1, DM), jnp.float32).at[0, :H].set(1.0)

    watt_s = Watt * (pri / jnp.sqrt(jnp.float32(DH)))[:, :, :, None, None]
    wk4 = Wk.reshape(L, T, DM, H, DH)
    wv4 = Wv.reshape(L, T, DM, H, DH)
    wkr = jnp.einsum('ltdhe,lhref->ltrdhf', wk4, watt_s).reshape(L, T, R, DM, DM)
    wvr = jnp.einsum('ltdhe,lhref->ltrdhf', wv4, Wmsg).reshape(L, T, R, DM, DM)

    h = _sc_gather_rows(embed.reshape(T * N, DM), flatidx)

    for l in range(L):
        q, krcat, vrcat = _dense1(h, ohf, Wq[l], wkr[l], wvr[l])
        kr = krcat.reshape(NROW * R, DM)
        vr = vrcat.reshape(NROW * R, DM)
        kre, qe, vre = _sc_gather3(srcet, dste, kr, q, vr)
        aexp, msg = _edgef(kre, qe, vre, repc, mask8, repr_)
        denp = _sc_scatter128(dste, aexp, zeros128)
        haggp = _sc_scatter128(dste, msg, zeros128)
        h = _dense2(haggp[0], haggp[1], denp[0], denp[1], repr_, h, ohf,
                    alcol[l], Wa[l],
                    ln_g[l].reshape(1, DM), ln_b[l].reshape(1, DM))
    return h[:N]
